# baseline (speedup 1.0000x reference)
import functools

import jax
import jax.numpy as jnp
from jax import lax
from jax.experimental import pallas as pl
from jax.experimental.pallas import tpu as pltpu
from jax.experimental.pallas import tpu_sc as plsc

D_MODEL = 768
LANES = 16
VPR = D_MODEL // LANES
NUM_CORES = 2
NUM_SUBCORES = 16
NW = NUM_CORES * NUM_SUBCORES
CHUNK = 64


def _emb_body(n_chunks, seq_len, tok_hbm, pos_hbm, idx_hbm, out_hbm,
              idx_v, rows_v, pos_v):
    cid = lax.axis_index("c")
    sid = lax.axis_index("s")
    wid = sid * NUM_CORES + cid
    base = wid * n_chunks
    pltpu.sync_copy(idx_hbm.at[pl.ds(base, n_chunks)], idx_v)
    t0 = (wid * n_chunks * CHUNK) % seq_len

    def add_body(i, _):
        r = i // VPR
        c = (i % VPR) * LANES
        rows_v[r, pl.ds(c, LANES)] = (rows_v[r, pl.ds(c, LANES)]
                                      + pos_v[r, pl.ds(c, LANES)])
        return 0

    for k in range(n_chunks):
        pltpu.sync_copy(tok_hbm.at[idx_v.at[k]], rows_v)
        pltpu.sync_copy(pos_hbm.at[pl.ds(t0 + k * CHUNK, CHUNK)], pos_v)
        lax.fori_loop(0, CHUNK * VPR, add_body, 0)
        pltpu.sync_copy(rows_v, out_hbm.at[pl.ds((base + k) * CHUNK, CHUNK)])


@jax.jit
def kernel(x, token_table, pos_table):
    B, T = x.shape
    n_rows = B * T
    n_chunks = n_rows // (NW * CHUNK)

    idx = x.astype(jnp.int32).reshape(NW * n_chunks, CHUNK)

    mesh = plsc.VectorSubcoreMesh(
        core_axis_name="c", subcore_axis_name="s")
    run = pl.kernel(
        functools.partial(_emb_body, n_chunks, T),
        out_type=jax.ShapeDtypeStruct((n_rows, D_MODEL), jnp.float32),
        mesh=mesh,
        scratch_types=[
            pltpu.VMEM((n_chunks, CHUNK), jnp.int32),
            pltpu.VMEM((CHUNK, D_MODEL), jnp.float32),
            pltpu.VMEM((CHUNK, D_MODEL), jnp.float32),
        ],
    )
    out = run(token_table, pos_table, idx)
    return out.reshape(B, T, D_MODEL)


# speedup vs baseline: 1.0017x; 1.0017x over previous
import functools

import jax
import jax.numpy as jnp
from jax import lax
from jax.experimental import pallas as pl
from jax.experimental.pallas import tpu as pltpu
from jax.experimental.pallas import tpu_sc as plsc

D_MODEL = 768
LANES = 16
VPR = D_MODEL // LANES
NUM_CORES = 2
NUM_SUBCORES = 16
NW = NUM_CORES * NUM_SUBCORES
CHUNK = 64


def _emb_body(n_chunks, seq_len, tok_hbm, pos_hbm, idx_hbm, out_hbm,
              idx_v, rows_v, pos_v):
    cid = lax.axis_index("c")
    sid = lax.axis_index("s")
    wid = sid * NUM_CORES + cid
    base = wid * n_chunks
    pltpu.sync_copy(idx_hbm.at[pl.ds(base, n_chunks)], idx_v)
    t0 = (wid * n_chunks * CHUNK) % seq_len

    def add_body(i, _):
        r = i // VPR
        c = (i % VPR) * LANES
        plsc.addupdate(rows_v.at[r, pl.ds(c, LANES)],
                       pos_v[r, pl.ds(c, LANES)])
        return 0

    for k in range(n_chunks):
        pltpu.sync_copy(tok_hbm.at[idx_v.at[k]], rows_v)
        pltpu.sync_copy(pos_hbm.at[pl.ds(t0 + k * CHUNK, CHUNK)], pos_v)
        lax.fori_loop(0, CHUNK * VPR, add_body, 0)
        pltpu.sync_copy(rows_v, out_hbm.at[pl.ds((base + k) * CHUNK, CHUNK)])


@jax.jit
def kernel(x, token_table, pos_table):
    B, T = x.shape
    n_rows = B * T
    n_chunks = n_rows // (NW * CHUNK)

    idx = x.astype(jnp.int32).reshape(NW * n_chunks, CHUNK)

    mesh = plsc.VectorSubcoreMesh(
        core_axis_name="c", subcore_axis_name="s")
    run = pl.kernel(
        functools.partial(_emb_body, n_chunks, T),
        out_type=jax.ShapeDtypeStruct((n_rows, D_MODEL), jnp.float32),
        mesh=mesh,
        scratch_types=[
            pltpu.VMEM((n_chunks, CHUNK), jnp.int32),
            pltpu.VMEM((CHUNK, D_MODEL), jnp.float32),
            pltpu.VMEM((CHUNK, D_MODEL), jnp.float32),
        ],
    )
    out = run(token_table, pos_table, idx)
    return out.reshape(B, T, D_MODEL)


# speedup vs baseline: 1.5728x; 1.5701x over previous
import functools

import jax
import jax.numpy as jnp
from jax import lax
from jax.experimental import pallas as pl
from jax.experimental.pallas import tpu as pltpu
from jax.experimental.pallas import tpu_sc as plsc

D_MODEL = 768
LANES = 16
VPR = D_MODEL // LANES
NUM_CORES = 2
NUM_SUBCORES = 16
NW = NUM_CORES * NUM_SUBCORES
CHUNK = 64


def _emb_body(n_chunks, seq_len, tok_hbm, pos_hbm, idx_hbm, out_hbm,
              idx_v, rows_v, pos_v):
    cid = lax.axis_index("c")
    sid = lax.axis_index("s")
    wid = sid * NUM_CORES + cid
    base = wid * n_chunks
    pltpu.sync_copy(idx_hbm.at[pl.ds(base, n_chunks)], idx_v)
    t0 = (wid * n_chunks * CHUNK) % seq_len

    def add_body(r, _):
        for c in range(VPR):
            plsc.addupdate(rows_v.at[r, pl.ds(c * LANES, LANES)],
                           pos_v[r, pl.ds(c * LANES, LANES)])
        return 0

    for k in range(n_chunks):
        pltpu.sync_copy(tok_hbm.at[idx_v.at[k]], rows_v)
        pltpu.sync_copy(pos_hbm.at[pl.ds(t0 + k * CHUNK, CHUNK)], pos_v)
        lax.fori_loop(0, CHUNK, add_body, 0)
        pltpu.sync_copy(rows_v, out_hbm.at[pl.ds((base + k) * CHUNK, CHUNK)])


@jax.jit
def kernel(x, token_table, pos_table):
    B, T = x.shape
    n_rows = B * T
    n_chunks = n_rows // (NW * CHUNK)

    idx = x.astype(jnp.int32).reshape(NW * n_chunks, CHUNK)

    mesh = plsc.VectorSubcoreMesh(
        core_axis_name="c", subcore_axis_name="s")
    run = pl.kernel(
        functools.partial(_emb_body, n_chunks, T),
        out_type=jax.ShapeDtypeStruct((n_rows, D_MODEL), jnp.float32),
        mesh=mesh,
        scratch_types=[
            pltpu.VMEM((n_chunks, CHUNK), jnp.int32),
            pltpu.VMEM((CHUNK, D_MODEL), jnp.float32),
            pltpu.VMEM((CHUNK, D_MODEL), jnp.float32),
        ],
    )
    out = run(token_table, pos_table, idx)
    return out.reshape(B, T, D_MODEL)


# speedup vs baseline: 1.9621x; 1.2475x over previous
import functools

import jax
import jax.numpy as jnp
from jax import lax
from jax.experimental import pallas as pl
from jax.experimental.pallas import tpu as pltpu
from jax.experimental.pallas import tpu_sc as plsc

D_MODEL = 768
LANES = 16
VPR = D_MODEL // LANES
NUM_CORES = 2
NUM_SUBCORES = 16
NW = NUM_CORES * NUM_SUBCORES
CHUNK = 32


def _emb_body(n_chunks, seq_len, tok_hbm, pos_hbm, idx_hbm, out_hbm,
              idx_v, rows0, rows1, pos0, pos1,
              sg0, sg1, sp0, sp1, ss0, ss1):
    cid = lax.axis_index("c")
    sid = lax.axis_index("s")
    wid = sid * NUM_CORES + cid
    base = wid * n_chunks
    pltpu.sync_copy(idx_hbm.at[pl.ds(base, n_chunks)], idx_v)
    t0 = (wid * n_chunks * CHUNK) % seq_len

    rows = [rows0, rows1]
    pos = [pos0, pos1]
    sg = [sg0, sg1]
    sp = [sp0, sp1]
    ss = [ss0, ss1]
    gd = [None, None]
    pd = [None, None]
    sd = [None, None]

    def make_add(rows_v, pos_v):
        def add_body(r, _):
            for c in range(VPR):
                plsc.addupdate(rows_v.at[r, pl.ds(c * LANES, LANES)],
                               pos_v[r, pl.ds(c * LANES, LANES)])
            return 0
        return add_body

    def start(k):
        b = k % 2
        gd[b] = pltpu.async_copy(tok_hbm.at[idx_v.at[k]], rows[b], sg[b])
        pd[b] = pltpu.async_copy(
            pos_hbm.at[pl.ds(t0 + k * CHUNK, CHUNK)], pos[b], sp[b])

    start(0)
    for k in range(n_chunks):
        b = k % 2
        gd[b].wait()
        pd[b].wait()
        if k + 1 < n_chunks:
            if k >= 1:
                sd[(k + 1) % 2].wait()
            start(k + 1)
        lax.fori_loop(0, CHUNK, make_add(rows[b], pos[b]), 0)
        sd[b] = pltpu.async_copy(
            rows[b], out_hbm.at[pl.ds((base + k) * CHUNK, CHUNK)], ss[b])
    sd[0].wait()
    sd[1].wait()


@jax.jit
def kernel(x, token_table, pos_table):
    B, T = x.shape
    n_rows = B * T
    n_chunks = n_rows // (NW * CHUNK)

    idx = x.astype(jnp.int32).reshape(NW * n_chunks, CHUNK)

    mesh = plsc.VectorSubcoreMesh(
        core_axis_name="c", subcore_axis_name="s")
    run = pl.kernel(
        functools.partial(_emb_body, n_chunks, T),
        out_type=jax.ShapeDtypeStruct((n_rows, D_MODEL), jnp.float32),
        mesh=mesh,
        scratch_types=[
            pltpu.VMEM((n_chunks, CHUNK), jnp.int32),
            pltpu.VMEM((CHUNK, D_MODEL), jnp.float32),
            pltpu.VMEM((CHUNK, D_MODEL), jnp.float32),
            pltpu.VMEM((CHUNK, D_MODEL), jnp.float32),
            pltpu.VMEM((CHUNK, D_MODEL), jnp.float32),
            pltpu.SemaphoreType.DMA,
            pltpu.SemaphoreType.DMA,
            pltpu.SemaphoreType.DMA,
            pltpu.SemaphoreType.DMA,
            pltpu.SemaphoreType.DMA,
            pltpu.SemaphoreType.DMA,
        ],
    )
    out = run(token_table, pos_table, idx)
    return out.reshape(B, T, D_MODEL)
